# baseline (speedup 1.0000x reference)
import functools

import jax
import jax.numpy as jnp
from jax import lax
from jax.experimental import pallas as pl
from jax.experimental.pallas import tpu as pltpu
from jax.experimental.pallas import tpu_sc as plsc

N = 10000
F = 16
FD = 8
NC = 2
NS = 16
NW = NC * NS
ROWS_PER_TILE = N // NS
C = 125
CH = 80
NB = 4
E = 320000
TC_BLK = 2000
_SC_PARAMS = pltpu.CompilerParams(use_tc_tiling_on_sc=False)


def _zero_acc(zeros_hbm, acc, s):
    base = s * ROWS_PER_TILE
    pltpu.sync_copy(zeros_hbm.at[pl.ds(base, ROWS_PER_TILE)],
                    acc.at[pl.ds(base, ROWS_PER_TILE)])


def _writeback(out_hbm, acc, c, s):
    base = s * ROWS_PER_TILE
    pltpu.sync_copy(acc.at[pl.ds(base, ROWS_PER_TILE)],
                    out_hbm.at[c, pl.ds(base, ROWS_PER_TILE)])


@functools.partial(
    pl.kernel,
    out_type=jax.ShapeDtypeStruct((NC, N, FD), jnp.float32),
    mesh=plsc.VectorSubcoreMesh(core_axis_name="c", subcore_axis_name="s"),
    compiler_params=_SC_PARAMS,
    scratch_types=[
        pltpu.VMEM_SHARED((N, FD), jnp.float32),
        pltpu.VMEM((C, FD), jnp.float32),
        pltpu.VMEM((CH, C), jnp.int32),
        pltpu.SemaphoreType.DMA,
        [pltpu.SemaphoreType.DMA] * NB,
    ],
)
def _deg_kernel(dst_hbm, zeros_hbm, ones_hbm, out_hbm, acc, ones_v, dstb, sem_e, sems):
    c = lax.axis_index("c")
    s = lax.axis_index("s")
    wid = s * NC + c
    ld = pltpu.async_copy(dst_hbm.at[wid], dstb, sem_e)
    _zero_acc(zeros_hbm, acc, s)
    pltpu.sync_copy(ones_hbm, ones_v)
    plsc.subcore_barrier()
    ld.wait()

    def grp(t, _):
        for p in range(NB):
            k = NB * t + p

            @pl.when(t > 0)
            def _():
                pltpu.make_async_copy(ones_v, acc.at[dstb.at[k - NB]], sems[p]).wait()

            pltpu.async_copy(ones_v, acc.at[dstb.at[k]], sems[p], add=True)
        return ()

    lax.fori_loop(0, CH // NB, grp, ())
    for p in range(NB):
        pltpu.make_async_copy(ones_v, acc.at[dstb.at[CH - NB + p]], sems[p]).wait()
    plsc.subcore_barrier()
    _writeback(out_hbm, acc, c, s)


def _make_edge_kernel(W):
    @functools.partial(
        pl.kernel,
        out_type=jax.ShapeDtypeStruct((NC, N, W), jnp.float32),
        mesh=plsc.VectorSubcoreMesh(core_axis_name="c", subcore_axis_name="s"),
        compiler_params=_SC_PARAMS,
        scratch_types=[
            pltpu.VMEM_SHARED((N, W), jnp.float32),
            pltpu.VMEM_SHARED((N, W), jnp.float32),
            [pltpu.VMEM((C, W), jnp.float32)] * NB,
            pltpu.VMEM((CH, C), jnp.int32),
            pltpu.VMEM((CH, C), jnp.int32),
            pltpu.SemaphoreType.DMA,
            pltpu.SemaphoreType.DMA,
            [pltpu.SemaphoreType.DMA] * NB,
            [pltpu.SemaphoreType.DMA] * NB,
        ],
    )
    def edge_kernel(table_hbm, zeros_hbm, src_hbm, dst_hbm, out_hbm,
                    acc, tbl, rows, srcb, dstb, sem_s, sem_d, semg, sems):
        c = lax.axis_index("c")
        s = lax.axis_index("s")
        wid = s * NC + c
        base = s * ROWS_PER_TILE
        lds = pltpu.async_copy(src_hbm.at[wid], srcb, sem_s)
        ldd = pltpu.async_copy(dst_hbm.at[wid], dstb, sem_d)
        _zero_acc(zeros_hbm, acc, s)
        pltpu.sync_copy(table_hbm.at[pl.ds(base, ROWS_PER_TILE)],
                        tbl.at[pl.ds(base, ROWS_PER_TILE)])
        plsc.subcore_barrier()
        lds.wait()
        ldd.wait()

        def grp(t, _):
            for p in range(NB):
                k = NB * t + p

                @pl.when(t > 0)
                def _():
                    pltpu.make_async_copy(rows[p], acc.at[dstb.at[k - NB]], sems[p]).wait()

                pltpu.async_copy(tbl.at[srcb.at[k]], rows[p], semg[p])
            for p in range(NB):
                k = NB * t + p
                pltpu.make_async_copy(tbl.at[srcb.at[k]], rows[p], semg[p]).wait()
                pltpu.async_copy(rows[p], acc.at[dstb.at[k]], sems[p], add=True)
            return ()

        lax.fori_loop(0, CH // NB, grp, ())
        for p in range(NB):
            pltpu.make_async_copy(rows[p], acc.at[dstb.at[CH - NB + p]], sems[p]).wait()
        plsc.subcore_barrier()
        _writeback(out_hbm, acc, c, s)

    return edge_kernel


_edge_kernel = _make_edge_kernel(F)
_edge_kernel8 = _make_edge_kernel(FD)


def _tc1_body(x_ref, w1_ref, degp_ref, h1s_ref, dis_ref):
    deg = degp_ref[0, :, 0] + degp_ref[1, :, 0] + 1.0
    dis = lax.rsqrt(deg)
    h = jnp.dot(x_ref[...], w1_ref[...], preferred_element_type=jnp.float32)
    dis16 = jnp.broadcast_to(dis[:, None], (TC_BLK, F))
    h1s_ref[...] = h * dis16
    dis_ref[...] = dis16


def _tc2_body(p_ref, h1s_ref, dis_ref, b1_ref, w2_ref, out_ref):
    dis16 = dis_ref[...]
    a = p_ref[0] + p_ref[1] + h1s_ref[...]
    h = jnp.maximum(dis16 * a + b1_ref[...], 0.0)
    h2 = jnp.dot(h, w2_ref[...], preferred_element_type=jnp.float32)
    out_ref[...] = h2 * dis16[:, :FD]


def _tc3_body(p_ref, h2s_ref, dis_ref, b2_ref, out_ref):
    z = dis_ref[:, :FD] * (p_ref[0] + p_ref[1] + h2s_ref[...]) + b2_ref[...]
    mask = lax.broadcasted_iota(jnp.int32, z.shape, 1) < 7
    zm = jnp.where(mask, z, -jnp.inf)
    m = jnp.max(zm, axis=1, keepdims=True)
    e = jnp.where(mask, jnp.exp(zm - m), 0.0)
    lse = jnp.log(jnp.sum(e, axis=1, keepdims=True))
    out_ref[...] = ((z - m) - lse)[:, :7]


def kernel(x, edge_index, W1, b1, W2, b2):
    f32 = jnp.float32
    ei = edge_index.astype(jnp.int32)
    src = ei[0].reshape(NW, CH, C)
    dst = ei[1].reshape(NW, CH, C)
    zeros16 = jnp.zeros((N, F), dtype=f32)
    zeros8 = jnp.zeros((N, FD), dtype=f32)
    ones8 = jnp.ones((C, FD), dtype=f32)
    W2p = jnp.zeros((F, FD), dtype=f32).at[:, :7].set(W2)
    b1p = jnp.broadcast_to(b1[None, :], (1, F))
    b2p = jnp.zeros((1, FD), dtype=f32).at[0, :7].set(b2)

    degp = _deg_kernel(dst, zeros8, ones8)

    grid = N // TC_BLK
    blk = lambda i: (i, 0)
    h1s, dis16 = pl.pallas_call(
        _tc1_body,
        grid=(grid,),
        in_specs=[
            pl.BlockSpec((TC_BLK, 128), blk),
            pl.BlockSpec((128, F), lambda i: (0, 0)),
            pl.BlockSpec((NC, TC_BLK, FD), lambda i: (0, i, 0)),
        ],
        out_specs=[pl.BlockSpec((TC_BLK, F), blk)] * 2,
        out_shape=[jax.ShapeDtypeStruct((N, F), f32)] * 2,
    )(x, W1, degp)

    p1 = _edge_kernel(h1s, zeros16, src, dst)

    h2s = pl.pallas_call(
        _tc2_body,
        grid=(1,),
        in_specs=[
            pl.BlockSpec((NC, N, F), lambda i: (0, 0, 0)),
            pl.BlockSpec((N, F), blk),
            pl.BlockSpec((N, F), blk),
            pl.BlockSpec((1, F), lambda i: (0, 0)),
            pl.BlockSpec((F, FD), lambda i: (0, 0)),
        ],
        out_specs=pl.BlockSpec((N, FD), blk),
        out_shape=jax.ShapeDtypeStruct((N, FD), f32),
    )(p1, h1s, dis16, b1p, W2p)

    p2 = _edge_kernel8(h2s, zeros8, src, dst)

    out = pl.pallas_call(
        _tc3_body,
        grid=(1,),
        in_specs=[
            pl.BlockSpec((NC, N, FD), lambda i: (0, 0, 0)),
            pl.BlockSpec((N, FD), blk),
            pl.BlockSpec((N, F), blk),
            pl.BlockSpec((1, FD), lambda i: (0, 0)),
        ],
        out_specs=pl.BlockSpec((N, 7), blk),
        out_shape=jax.ShapeDtypeStruct((N, 7), f32),
    )(p2, h2s, dis16, b2p)

    return out


# speedup vs baseline: 1.0273x; 1.0273x over previous
import functools

import jax
import jax.numpy as jnp
from jax import lax
from jax.experimental import pallas as pl
from jax.experimental.pallas import tpu as pltpu
from jax.experimental.pallas import tpu_sc as plsc

N = 10000
F = 16
FD = 8
NC = 2
NS = 16
NW = NC * NS
ROWS_PER_TILE = N // NS
C = 125
CH = 80
NB = 4
E = 320000
TC_BLK = 5000
_SC_PARAMS = pltpu.CompilerParams(use_tc_tiling_on_sc=False)


def _zero_acc(zeros_hbm, acc, s):
    base = s * ROWS_PER_TILE
    pltpu.sync_copy(zeros_hbm.at[pl.ds(base, ROWS_PER_TILE)],
                    acc.at[pl.ds(base, ROWS_PER_TILE)])


def _writeback(out_hbm, acc, c, s):
    base = s * ROWS_PER_TILE
    pltpu.sync_copy(acc.at[pl.ds(base, ROWS_PER_TILE)],
                    out_hbm.at[c, pl.ds(base, ROWS_PER_TILE)])


@functools.partial(
    pl.kernel,
    out_type=jax.ShapeDtypeStruct((NC, N, FD), jnp.float32),
    mesh=plsc.VectorSubcoreMesh(core_axis_name="c", subcore_axis_name="s"),
    compiler_params=_SC_PARAMS,
    scratch_types=[
        pltpu.VMEM_SHARED((N, FD), jnp.float32),
        pltpu.VMEM((C, FD), jnp.float32),
        pltpu.VMEM((CH, C), jnp.int32),
        pltpu.SemaphoreType.DMA,
        [pltpu.SemaphoreType.DMA] * NB,
    ],
)
def _deg_kernel(dst_hbm, zeros_hbm, ones_hbm, out_hbm, acc, ones_v, dstb, sem_e, sems):
    c = lax.axis_index("c")
    s = lax.axis_index("s")
    wid = s * NC + c
    ld = pltpu.async_copy(dst_hbm.at[wid], dstb, sem_e)
    _zero_acc(zeros_hbm, acc, s)
    pltpu.sync_copy(ones_hbm, ones_v)
    plsc.subcore_barrier()
    ld.wait()

    def grp(t, _):
        for p in range(NB):
            k = NB * t + p

            @pl.when(t > 0)
            def _():
                pltpu.make_async_copy(ones_v, acc.at[dstb.at[k - NB]], sems[p]).wait()

            pltpu.async_copy(ones_v, acc.at[dstb.at[k]], sems[p], add=True)
        return ()

    lax.fori_loop(0, CH // NB, grp, ())
    for p in range(NB):
        pltpu.make_async_copy(ones_v, acc.at[dstb.at[CH - NB + p]], sems[p]).wait()
    plsc.subcore_barrier()
    _writeback(out_hbm, acc, c, s)


def _make_edge_kernel(W):
    @functools.partial(
        pl.kernel,
        out_type=jax.ShapeDtypeStruct((NC, N, W), jnp.float32),
        mesh=plsc.VectorSubcoreMesh(core_axis_name="c", subcore_axis_name="s"),
        compiler_params=_SC_PARAMS,
        scratch_types=[
            pltpu.VMEM_SHARED((N, W), jnp.float32),
            pltpu.VMEM_SHARED((N, W), jnp.float32),
            [pltpu.VMEM((C, W), jnp.float32)] * NB,
            pltpu.VMEM((CH, C), jnp.int32),
            pltpu.VMEM((CH, C), jnp.int32),
            pltpu.SemaphoreType.DMA,
            pltpu.SemaphoreType.DMA,
            [pltpu.SemaphoreType.DMA] * NB,
            [pltpu.SemaphoreType.DMA] * NB,
        ],
    )
    def edge_kernel(table_hbm, zeros_hbm, src_hbm, dst_hbm, out_hbm,
                    acc, tbl, rows, srcb, dstb, sem_s, sem_d, semg, sems):
        c = lax.axis_index("c")
        s = lax.axis_index("s")
        wid = s * NC + c
        base = s * ROWS_PER_TILE
        lds = pltpu.async_copy(src_hbm.at[wid], srcb, sem_s)
        ldd = pltpu.async_copy(dst_hbm.at[wid], dstb, sem_d)
        _zero_acc(zeros_hbm, acc, s)
        pltpu.sync_copy(table_hbm.at[pl.ds(base, ROWS_PER_TILE)],
                        tbl.at[pl.ds(base, ROWS_PER_TILE)])
        plsc.subcore_barrier()
        lds.wait()
        ldd.wait()

        def grp(t, _):
            for p in range(NB):
                k = NB * t + p

                @pl.when(t > 0)
                def _():
                    pltpu.make_async_copy(rows[p], acc.at[dstb.at[k - NB]], sems[p]).wait()

                pltpu.async_copy(tbl.at[srcb.at[k]], rows[p], semg[p])
            for p in range(NB):
                k = NB * t + p
                pltpu.make_async_copy(tbl.at[srcb.at[k]], rows[p], semg[p]).wait()
                pltpu.async_copy(rows[p], acc.at[dstb.at[k]], sems[p], add=True)
            return ()

        lax.fori_loop(0, CH // NB, grp, ())
        for p in range(NB):
            pltpu.make_async_copy(rows[p], acc.at[dstb.at[CH - NB + p]], sems[p]).wait()
        plsc.subcore_barrier()
        _writeback(out_hbm, acc, c, s)

    return edge_kernel


_edge_kernel = _make_edge_kernel(F)
_edge_kernel8 = _make_edge_kernel(FD)


def _tc1_body(x_ref, w1_ref, degp_ref, h1s_ref, dis_ref):
    deg = degp_ref[0, :, 0] + degp_ref[1, :, 0] + 1.0
    dis = lax.rsqrt(deg)
    h = jnp.dot(x_ref[...], w1_ref[...], preferred_element_type=jnp.float32)
    dis16 = jnp.broadcast_to(dis[:, None], (TC_BLK, F))
    h1s_ref[...] = h * dis16
    dis_ref[...] = dis16


def _tc2_body(p_ref, h1s_ref, dis_ref, b1_ref, w2_ref, out_ref):
    dis16 = dis_ref[...]
    a = p_ref[0] + p_ref[1] + h1s_ref[...]
    h = jnp.maximum(dis16 * a + b1_ref[...], 0.0)
    h2 = jnp.dot(h, w2_ref[...], preferred_element_type=jnp.float32)
    out_ref[...] = h2 * dis16[:, :FD]


def _tc3_body(p_ref, h2s_ref, dis_ref, b2_ref, out_ref):
    z = dis_ref[:, :FD] * (p_ref[0] + p_ref[1] + h2s_ref[...]) + b2_ref[...]
    mask = lax.broadcasted_iota(jnp.int32, (TC_BLK, FD), 1) < 7
    zm = jnp.where(mask, z, -jnp.inf)
    m = jnp.max(zm, axis=1, keepdims=True)
    e = jnp.where(mask, jnp.exp(zm - m), 0.0)
    lse = jnp.log(jnp.sum(e, axis=1, keepdims=True))
    out_ref[...] = ((z - m) - lse)[:, :7]


def kernel(x, edge_index, W1, b1, W2, b2):
    f32 = jnp.float32
    ei = edge_index.astype(jnp.int32)
    src = ei[0].reshape(NW, CH, C)
    dst = ei[1].reshape(NW, CH, C)
    zeros16 = jnp.zeros((N, F), dtype=f32)
    zeros8 = jnp.zeros((N, FD), dtype=f32)
    ones8 = jnp.ones((C, FD), dtype=f32)
    W2p = jnp.zeros((F, FD), dtype=f32).at[:, :7].set(W2)
    b1p = jnp.broadcast_to(b1[None, :], (1, F))
    b2p = jnp.zeros((1, FD), dtype=f32).at[0, :7].set(b2)

    degp = _deg_kernel(dst, zeros8, ones8)

    grid = N // TC_BLK
    blk = lambda i: (i, 0)
    h1s, dis16 = pl.pallas_call(
        _tc1_body,
        grid=(grid,),
        in_specs=[
            pl.BlockSpec((TC_BLK, 128), blk),
            pl.BlockSpec((128, F), lambda i: (0, 0)),
            pl.BlockSpec((NC, TC_BLK, FD), lambda i: (0, i, 0)),
        ],
        out_specs=[pl.BlockSpec((TC_BLK, F), blk)] * 2,
        out_shape=[jax.ShapeDtypeStruct((N, F), f32)] * 2,
    )(x, W1, degp)

    p1 = _edge_kernel(h1s, zeros16, src, dst)

    h2s = pl.pallas_call(
        _tc2_body,
        grid=(grid,),
        in_specs=[
            pl.BlockSpec((NC, TC_BLK, F), lambda i: (0, i, 0)),
            pl.BlockSpec((TC_BLK, F), blk),
            pl.BlockSpec((TC_BLK, F), blk),
            pl.BlockSpec((1, F), lambda i: (0, 0)),
            pl.BlockSpec((F, FD), lambda i: (0, 0)),
        ],
        out_specs=pl.BlockSpec((TC_BLK, FD), blk),
        out_shape=jax.ShapeDtypeStruct((N, FD), f32),
    )(p1, h1s, dis16, b1p, W2p)

    p2 = _edge_kernel8(h2s, zeros8, src, dst)

    out = pl.pallas_call(
        _tc3_body,
        grid=(grid,),
        in_specs=[
            pl.BlockSpec((NC, TC_BLK, FD), lambda i: (0, i, 0)),
            pl.BlockSpec((TC_BLK, FD), blk),
            pl.BlockSpec((TC_BLK, F), blk),
            pl.BlockSpec((1, FD), lambda i: (0, 0)),
        ],
        out_specs=pl.BlockSpec((TC_BLK, 7), blk),
        out_shape=jax.ShapeDtypeStruct((N, 7), f32),
    )(p2, h2s, dis16, b2p)

    return out
